# Initial kernel scaffold; baseline (speedup 1.0000x reference)
import jax
import jax.numpy as jnp
from jax.experimental import pallas as pl


def kernel(X, A, Wl0, Wr0, att0, b0, Wl1, Wr1, att1, b1, Wl2, Wr2, att2, b2):
    raise NotImplementedError("write your pallas kernel here")


# speedup vs baseline: 61.2790x; 61.2790x over previous
import functools

import jax
import jax.numpy as jnp
from jax.experimental import pallas as pl


_BD = 128


def _prep_body(x_ref, wl_ref, wr_ref, xl_ref, xlT_ref, xrT_ref):
    xv = x_ref[...]
    norm = jnp.sqrt(jnp.sum(xv * xv, axis=1, keepdims=True))
    xn = xv / (norm + 1e-12)
    wl = wl_ref[...]
    wr = wr_ref[...]
    xl_ref[...] = jax.lax.dot_general(
        xn, wl, (((1,), (0,)), ((), ())), preferred_element_type=jnp.float32)
    xlT_ref[...] = jax.lax.dot_general(
        wl, xn, (((0,), (1,)), ((), ())), preferred_element_type=jnp.float32)
    xrT_ref[...] = jax.lax.dot_general(
        wr, xn, (((0,), (1,)), ((), ())), preferred_element_type=jnp.float32)


def _att_body(at_ref, xlT_ref, xrT_ref, xl_ref, att_ref, b_ref, o_ref, *, f_out, n):
    mask = at_ref[...] != 0.0
    unroll = 4

    def step(dd, e):
        for k in range(unroll):
            d = dd * unroll + k
            row = xlT_ref[d]
            col = xrT_ref[d]
            a_d = att_ref[d]
            z = row + col
            e = e + a_d * jnp.maximum(z, 0.2 * z)
        return e

    e = jax.lax.fori_loop(
        0, f_out // unroll, step, jnp.zeros((_BD, n), jnp.float32))

    neg_inf = jnp.float32(-jnp.inf)
    em = jnp.where(mask, e, neg_inf)
    m = jnp.max(em, axis=1, keepdims=True)
    mf = jnp.where(m > neg_inf, m, 0.0)
    ex = jnp.where(mask, jnp.exp(e - mf), 0.0)
    s = jnp.sum(ex, axis=1, keepdims=True)
    alpha = ex / (s + 1e-16)
    out = jax.lax.dot_general(
        alpha, xl_ref[...], (((1,), (0,)), ((), ())),
        preferred_element_type=jnp.float32)
    o_ref[...] = out + b_ref[...]


def _layer(h, at, Wl, Wr, att, b):
    n, f_in = h.shape
    f_out = Wl.shape[1]

    xl, xlT, xrT = pl.pallas_call(
        _prep_body,
        out_shape=[
            jax.ShapeDtypeStruct((n, f_out), jnp.float32),
            jax.ShapeDtypeStruct((f_out, n), jnp.float32),
            jax.ShapeDtypeStruct((f_out, n), jnp.float32),
        ],
    )(h, Wl, Wr)

    xlT3 = xlT.reshape(f_out, 1, n)
    xrT3 = xrT.reshape(f_out, n, 1)
    att3 = att.reshape(f_out, 1, 1)
    b2 = b.reshape(1, f_out)

    grid = n // _BD
    out = pl.pallas_call(
        functools.partial(_att_body, f_out=f_out, n=n),
        grid=(grid,),
        in_specs=[
            pl.BlockSpec((_BD, n), lambda i: (i, 0)),
            pl.BlockSpec((f_out, 1, n), lambda i: (0, 0, 0)),
            pl.BlockSpec((f_out, _BD, 1), lambda i: (0, i, 0)),
            pl.BlockSpec((n, f_out), lambda i: (0, 0)),
            pl.BlockSpec((f_out, 1, 1), lambda i: (0, 0, 0)),
            pl.BlockSpec((1, f_out), lambda i: (0, 0)),
        ],
        out_specs=pl.BlockSpec((_BD, f_out), lambda i: (i, 0)),
        out_shape=jax.ShapeDtypeStruct((n, f_out), jnp.float32),
    )(at, xlT3, xrT3, xl, att3, b2)
    return out


def kernel(X, A, Wl0, Wr0, att0, b0, Wl1, Wr1, att1, b1, Wl2, Wr2, att2, b2):
    at = jnp.swapaxes(A, 0, 1)
    h = X
    for (Wl, Wr, att, b) in ((Wl0, Wr0, att0, b0),
                             (Wl1, Wr1, att1, b1),
                             (Wl2, Wr2, att2, b2)):
        h = _layer(h, at, Wl, Wr, att, b)
    return h


# speedup vs baseline: 177.6300x; 2.8987x over previous
import functools

import jax
import jax.numpy as jnp
from jax import lax
from jax.experimental import pallas as pl
from jax.experimental.pallas import tpu as pltpu
from jax.experimental.pallas import tpu_sc as plsc

N = 4096
NC = 2
NS = 16
NW = NC * NS
ROWS_PER_W = N // NW
NBR_STRIDE = 4608
L = 16

_mesh = plsc.VectorSubcoreMesh(core_axis_name="c", subcore_axis_name="s")
_sc_params = pltpu.CompilerParams(needs_layout_passes=False)


def _wid():
    return lax.axis_index("s") * NC + lax.axis_index("c")


@functools.partial(
    pl.kernel,
    mesh=_mesh,
    out_type=[
        jax.ShapeDtypeStruct((N * NBR_STRIDE,), jnp.int32),
        jax.ShapeDtypeStruct((N,), jnp.int32),
    ],
    scratch_types=[
        pltpu.VMEM((N,), jnp.float32),
        pltpu.VMEM((NBR_STRIDE,), jnp.int32),
        pltpu.VMEM((ROWS_PER_W,), jnp.int32),
    ],
    compiler_params=_sc_params,
)
def _extract(at_hbm, nbr_hbm, deg_hbm, row_v, nbrrow_v, deg_v):
    wid = _wid()

    def do_row(rl, _):
        r = wid * ROWS_PER_W + rl
        pltpu.sync_copy(at_hbm.at[pl.ds(r * N, N)], row_v)

        def chunk(j, cnt):
            lane = lax.iota(jnp.int32, 16)
            izero = jnp.zeros((16,), jnp.int32)
            trash = jnp.full((16,), NBR_STRIDE - 1, jnp.int32)
            v = row_v[pl.ds(j * 16, 16)]
            m32 = (v != 0.0).astype(jnp.int32)
            vals = lane + (izero + j * 16)
            csum = plsc.cumsum(m32)
            pos = jnp.where(m32 != 0, (izero + cnt) + csum - 1, trash)
            plsc.store_scatter(nbrrow_v, [pos], vals)
            return cnt + jnp.sum(m32)

        cnt = lax.fori_loop(0, N // 16, chunk, jnp.int32(0))
        nbrrow_v[pl.ds(cnt, 16)] = jnp.zeros((16,), jnp.int32)
        plsc.store_scatter(deg_v, [jnp.zeros((16,), jnp.int32) + rl],
                           jnp.zeros((16,), jnp.int32) + cnt)
        nch = (cnt + 526) // 512

        def wb(c, _):
            pltpu.sync_copy(nbrrow_v.at[pl.ds(c * 512, 512)],
                            nbr_hbm.at[pl.ds(r * NBR_STRIDE + c * 512, 512)])
            return 0

        lax.fori_loop(0, nch, wb, 0)
        return 0

    lax.fori_loop(0, ROWS_PER_W, do_row, 0)
    pltpu.sync_copy(deg_v, deg_hbm.at[pl.ds(wid * ROWS_PER_W, ROWS_PER_W)])


def _prep_body(x_ref, wl_ref, wr_ref, xl_ref, xr_ref):
    xv = x_ref[...]
    norm = jnp.sqrt(jnp.sum(xv * xv, axis=1, keepdims=True))
    xn = xv / (norm + 1e-12)
    xl_ref[...] = jax.lax.dot_general(
        xn, wl_ref[...], (((1,), (0,)), ((), ())),
        preferred_element_type=jnp.float32)
    xr_ref[...] = jax.lax.dot_general(
        xn, wr_ref[...], (((1,), (0,)), ((), ())),
        preferred_element_type=jnp.float32)


def _prep(h, Wl, Wr):
    n, _ = h.shape
    f_out = Wl.shape[1]
    return pl.pallas_call(
        _prep_body,
        out_shape=[
            jax.ShapeDtypeStruct((n, f_out), jnp.float32),
            jax.ShapeDtypeStruct((n, f_out), jnp.float32),
        ],
    )(h, Wl, Wr)


def _make_att(F):
    T = F // 16

    @functools.partial(
        pl.kernel,
        mesh=_mesh,
        out_type=jax.ShapeDtypeStruct((N * F,), jnp.float32),
        scratch_types=[
            pltpu.VMEM((ROWS_PER_W * F,), jnp.float32),
            pltpu.VMEM((ROWS_PER_W * F,), jnp.float32),
            pltpu.VMEM((NBR_STRIDE,), jnp.int32),
            pltpu.VMEM((16, F), jnp.float32),
            pltpu.VMEM((F,), jnp.float32),
            pltpu.VMEM((F,), jnp.float32),
            pltpu.VMEM((ROWS_PER_W,), jnp.int32),
            pltpu.SemaphoreType.DMA,
        ],
        compiler_params=_sc_params,
    )
    def att_kernel(xl_hbm, xr_hbm, att_hbm, b_hbm, nbr_hbm, deg_hbm, out_hbm,
                   xr_v, hout_v, nbr_v, rows_v, att_v, b_v, deg_v, sem):
        wid = _wid()
        base_row = wid * ROWS_PER_W
        lane = lax.iota(jnp.int32, 16)
        pltpu.sync_copy(xr_hbm.at[pl.ds(base_row * F, ROWS_PER_W * F)], xr_v)
        pltpu.sync_copy(att_hbm, att_v)
        pltpu.sync_copy(b_hbm, b_v)
        pltpu.sync_copy(deg_hbm.at[pl.ds(base_row, ROWS_PER_W)], deg_v)
        att_l = [att_v[pl.ds(16 * t, 16)] for t in range(T)]
        b_l = [b_v[pl.ds(16 * t, 16)] for t in range(T)]
        zero = jnp.zeros((16,), jnp.float32)
        izero = jnp.zeros((16,), jnp.int32)
        onef = jnp.full((16,), 1.0, jnp.float32)
        slope = jnp.full((16,), 0.2, jnp.float32)
        epsv = jnp.full((16,), 1e-16, jnp.float32)

        def do_dst(rl, _):
            r = base_row + rl
            dgrp = deg_v[pl.ds((rl // 16) * 16, 16)]
            d_r = jnp.sum(jnp.where(lane == (izero + rl % 16), dgrp, izero))
            nch = (d_r + 526) // 512

            def ld_nbr(c, _):
                pltpu.sync_copy(
                    nbr_hbm.at[pl.ds(r * NBR_STRIDE + c * 512, 512)],
                    nbr_v.at[pl.ds(c * 512, 512)])
                return 0

            lax.fori_loop(0, nch, ld_nbr, 0)
            xr_l = [xr_v[pl.ds(rl * F + 16 * t, 16)] for t in range(T)]

            def chunk(ch, car):
                m, s, acc = car
                idx_ref = nbr_v.at[pl.ds(ch * 16, 16)]
                pltpu.async_copy(xl_hbm.at[idx_ref], rows_v, sem).wait()
                for k in range(16):
                    ta = zero
                    for t in range(T):
                        row_t = rows_v[k, pl.ds(16 * t, 16)]
                        z = row_t + xr_l[t]
                        lr = jnp.maximum(z, slope * z)
                        ta = ta + att_l[t] * lr
                    e = zero + jnp.sum(ta)
                    validv = (izero + (ch * 16 + k)) < (izero + d_r)
                    mn = jnp.where(validv, jnp.maximum(m, e), m)
                    c1 = jnp.exp(m - mn)
                    p = jnp.where(validv, jnp.exp(e - mn), zero)
                    s = s * c1 + p
                    acc = [acc[t] * c1 + p * rows_v[k, pl.ds(16 * t, 16)]
                           for t in range(T)]
                    m = mn
                return (m, s, acc)

            m0 = jnp.full((16,), -1e30, jnp.float32)
            m, s, acc = lax.fori_loop(0, (d_r + 15) // 16, chunk,
                                      (m0, zero, [zero] * T))
            inv = onef / (s + epsv)
            for t in range(T):
                hout_v[pl.ds(rl * F + 16 * t, 16)] = acc[t] * inv + b_l[t]
            return 0

        lax.fori_loop(0, ROWS_PER_W, do_dst, 0)
        pltpu.sync_copy(
            hout_v, out_hbm.at[pl.ds(base_row * F, ROWS_PER_W * F)])

    return att_kernel


def kernel(X, A, Wl0, Wr0, att0, b0, Wl1, Wr1, att1, b1, Wl2, Wr2, att2, b2):
    at_flat = jnp.swapaxes(A, 0, 1).reshape(-1)
    nbr, deg = _extract(at_flat)
    h = X
    for (Wl, Wr, att, b) in ((Wl0, Wr0, att0, b0),
                             (Wl1, Wr1, att1, b1),
                             (Wl2, Wr2, att2, b2)):
        f_out = Wl.shape[1]
        f_pad = max(f_out, 128)
        if f_pad != f_out:
            pad = ((0, 0), (0, f_pad - f_out))
            Wl = jnp.pad(Wl, pad)
            Wr = jnp.pad(Wr, pad)
            att = jnp.pad(att, (0, f_pad - f_out))
            b = jnp.pad(b, (0, f_pad - f_out))
        xl, xr = _prep(h, Wl, Wr)
        hf = _make_att(f_pad)(xl, xr.reshape(-1), att, b, nbr, deg)
        h = hf.reshape(N, f_pad)[:, :f_out]
    return h
